# baseline (speedup 1.0000x reference)
import functools

import jax
import jax.numpy as jnp
from jax import lax
from jax.experimental import pallas as pl
from jax.experimental.pallas import tpu as pltpu, tpu_sc as plsc

N = 128
B = 8192
K = 9
W = 5
C = W * W
L = 16
NC, NS = 2, 16
NW = NC * NS
QPW = B // NW
NG = QPW // L


def _nsqrt(z, iters):
    i = plsc.bitcast(z, jnp.int32)
    i = 0x5F3759DF - (i >> 1)
    y = plsc.bitcast(i, jnp.float32)
    z5 = 0.5 * z
    for _ in range(iters):
        y = y * (1.5 - z5 * y * y)
    return z * y


def _body(x0_hbm, x1_hbm, w_hbm, lin_hbm, out_hbm,
          x0_v, x1_v, w_v, lin_v, h0_v, h1_v, out_v, wsem):
    wid = lax.axis_index("s") * NC + lax.axis_index("c")
    base = wid * QPW
    wcopy = pltpu.async_copy(w_hbm, w_v, wsem)
    pltpu.sync_copy(x0_hbm.at[pl.ds(base, QPW)], x0_v)
    pltpu.sync_copy(x1_hbm.at[pl.ds(base, QPW)], x1_v)
    pltpu.sync_copy(lin_hbm, lin_v)
    pltpu.sync_copy(x0_hbm.at[pl.ds(0, L)], h0_v.at[pl.ds(0, L)])
    pltpu.sync_copy(x1_hbm.at[pl.ds(0, L)], h1_v.at[pl.ds(0, L)])

    lanes = lax.iota(jnp.int32, L)
    v0 = h0_v[pl.ds(0, L)]
    v1 = h1_v[pl.ds(0, L)]
    neg = jnp.float32(-3.4e38)
    p00 = jnp.max(jnp.where(lanes == 0, v0, neg), axis=0)
    p01 = jnp.max(jnp.where(lanes == 1, v0, neg), axis=0)
    p10 = jnp.max(jnp.where(lanes == 0, v1, neg), axis=0)
    p11 = jnp.max(jnp.where(lanes == 1, v1, neg), axis=0)
    d0 = p00 - p01
    d1 = p10 - p11
    h2 = d0 * d0 + d1 * d1
    invh2 = 1.0 / jnp.full((L,), h2, jnp.float32)
    fastp = h2 > jnp.float32(64.0 * (2.0 / 127.0) ** 2)
    wcopy.wait()

    def group_body(g, carry):
        qs = pl.ds(g * L, L)
        qx0 = x0_v[qs]
        qx1 = x1_v[qs]
        t0 = (qx0 + 1.0) * ((N - 1) / 2.0)
        t1 = (qx1 + 1.0) * ((N - 1) / 2.0)
        si = jnp.clip((t0 + 0.5).astype(jnp.int32) - 2, 0, N - W)
        sj = jnp.clip((t1 + 0.5).astype(jnp.int32) - 2, 0, N - W)
        A = []
        Bc = []
        for d in range(W):
            rl = plsc.load_gather(lin_v, [si + d])
            cl = plsc.load_gather(lin_v, [sj + d])
            ar = qx0 - rl
            ac = qx1 - cl
            A.append(ar * ar)
            Bc.append(ac * ac)
        D = [A[di] + Bc[dj] for di in range(W) for dj in range(W)]
        rpos = [[jnp.full((L,), C - 1 - c, jnp.int32)] for c in range(C)]
        rneg = [[] for _ in range(C)]
        for c in range(C):
            for cp in range(c + 1, C):
                le = (D[c] <= D[cp]).astype(jnp.int32)
                rpos[cp].append(le)
                rneg[c].append(le)

        def _tree(t):
            while len(t) > 1:
                t = [t[i] + t[i + 1] if i + 1 < len(t) else t[i]
                     for i in range(0, len(t), 2)]
            return t[0]

        rank = [_tree(rpos[c]) - _tree(rneg[c]) if rneg[c] else _tree(rpos[c])
                for c in range(C)]
        sibase = si * N
        wvs = [plsc.load_gather(w_v, [(sibase + di * N) + (sj + dj)])
               for di in range(W) for dj in range(W)]
        sels = [rank[c] < K for c in range(C)]

        def _accum(full):
            terms = []
            for c in range(C):
                z = D[c] * invh2
                a = _nsqrt(z, 2 if full else 1)
                f1 = (1.5 * a - 2.5) * z + 1.0
                if full:
                    f2 = (2.5 - 0.5 * a) * z + (2.0 - 4.0 * a)
                    conv = jnp.where(z < 1.0, f1,
                                     jnp.where((z > 1.0) & (z < 4.0), f2, 0.0))
                else:
                    conv = f1
                terms.append(jnp.where(sels[c], wvs[c] * conv, 0.0))
            while len(terms) > 1:
                terms = [terms[i] + terms[i + 1] if i + 1 < len(terms)
                         else terms[i] for i in range(0, len(terms), 2)]
            return terms[0]

        acc = lax.cond(fastp, lambda: _accum(False), lambda: _accum(True))
        out_v[qs] = acc
        return carry

    lax.fori_loop(0, NG, group_body, 0)
    pltpu.sync_copy(out_v, out_hbm.at[pl.ds(base, QPW)])


@jax.jit
def kernel(x, weights, control_points):
    x0 = x[:, 0]
    x1 = x[:, 1]
    wf = weights[:, 0]
    lin = control_points[:N, 1]
    mesh = plsc.VectorSubcoreMesh(core_axis_name="c", subcore_axis_name="s")
    fn = pl.kernel(
        _body,
        out_type=jax.ShapeDtypeStruct((B,), jnp.float32),
        mesh=mesh,
        compiler_params=pltpu.CompilerParams(needs_layout_passes=False),
        scratch_types=[
            pltpu.VMEM((QPW,), jnp.float32),
            pltpu.VMEM((QPW,), jnp.float32),
            pltpu.VMEM((N * N,), jnp.float32),
            pltpu.VMEM((N,), jnp.float32),
            pltpu.VMEM((N,), jnp.float32),
            pltpu.VMEM((N,), jnp.float32),
            pltpu.VMEM((QPW,), jnp.float32),
            pltpu.SemaphoreType.DMA,
        ],
    )
    out = fn(x0, x1, wf, lin)
    return (out[:, None], x)


# speedup vs baseline: 1.1463x; 1.1463x over previous
import functools

import jax
import jax.numpy as jnp
from jax import lax
from jax.experimental import pallas as pl
from jax.experimental.pallas import tpu as pltpu, tpu_sc as plsc

N = 128
B = 8192
K = 9
W = 5
C = W * W
L = 16
NC, NS = 2, 16
NW = NC * NS
QPW = B // NW
NG = QPW // L


def _nsqrt(z, iters):
    i = plsc.bitcast(z, jnp.int32)
    i = 0x5F3759DF - (i >> 1)
    y = plsc.bitcast(i, jnp.float32)
    z5 = 0.5 * z
    for _ in range(iters):
        y = y * (1.5 - z5 * y * y)
    return z * y


def _body(x0_hbm, x1_hbm, w_hbm, lin_hbm, out_hbm,
          x0_v, x1_v, w_v, lin_v, h0_v, h1_v, out_v, wsem):
    wid = lax.axis_index("s") * NC + lax.axis_index("c")
    base = wid * QPW
    wcopy = pltpu.async_copy(w_hbm, w_v, wsem)
    pltpu.sync_copy(x0_hbm.at[pl.ds(base, QPW)], x0_v)
    pltpu.sync_copy(x1_hbm.at[pl.ds(base, QPW)], x1_v)
    pltpu.sync_copy(lin_hbm, lin_v)
    pltpu.sync_copy(x0_hbm.at[pl.ds(0, L)], h0_v.at[pl.ds(0, L)])
    pltpu.sync_copy(x1_hbm.at[pl.ds(0, L)], h1_v.at[pl.ds(0, L)])

    lanes = lax.iota(jnp.int32, L)
    v0 = h0_v[pl.ds(0, L)]
    v1 = h1_v[pl.ds(0, L)]
    neg = jnp.float32(-3.4e38)
    p00 = jnp.max(jnp.where(lanes == 0, v0, neg), axis=0)
    p01 = jnp.max(jnp.where(lanes == 1, v0, neg), axis=0)
    p10 = jnp.max(jnp.where(lanes == 0, v1, neg), axis=0)
    p11 = jnp.max(jnp.where(lanes == 1, v1, neg), axis=0)
    d0 = p00 - p01
    d1 = p10 - p11
    h2 = d0 * d0 + d1 * d1
    invh2 = 1.0 / jnp.full((L,), h2, jnp.float32)
    fastp = h2 > jnp.float32(64.0 * (2.0 / 127.0) ** 2)
    wcopy.wait()

    def group_body(g):
        qs = pl.ds(g * L, L)
        qx0 = x0_v[qs]
        qx1 = x1_v[qs]
        t0 = (qx0 + 1.0) * ((N - 1) / 2.0)
        t1 = (qx1 + 1.0) * ((N - 1) / 2.0)
        si = jnp.clip((t0 + 0.5).astype(jnp.int32) - 2, 0, N - W)
        sj = jnp.clip((t1 + 0.5).astype(jnp.int32) - 2, 0, N - W)
        A = []
        Bc = []
        for d in range(W):
            rl = plsc.load_gather(lin_v, [si + d])
            cl = plsc.load_gather(lin_v, [sj + d])
            ar = qx0 - rl
            ac = qx1 - cl
            A.append(ar * ar)
            Bc.append(ac * ac)
        D = [A[di] + Bc[dj] for di in range(W) for dj in range(W)]
        rank = [jnp.full((L,), C - 1 - c, jnp.int32) for c in range(C)]
        for c in range(C):
            for cp in range(c + 1, C):
                le = (D[c] <= D[cp]).astype(jnp.int32)
                rank[cp] = rank[cp] + le
                rank[c] = rank[c] - le
        sibase = si * N
        wvs = [plsc.load_gather(w_v, [(sibase + di * N) + (sj + dj)])
               for di in range(W) for dj in range(W)]
        sels = [rank[c] < K for c in range(C)]

        def _accum(full):
            terms = []
            for c in range(C):
                z = D[c] * invh2
                a = _nsqrt(z, 2 if full else 1)
                f1 = (1.5 * a - 2.5) * z + 1.0
                if full:
                    f2 = (2.5 - 0.5 * a) * z + (2.0 - 4.0 * a)
                    conv = jnp.where(z < 1.0, f1,
                                     jnp.where((z > 1.0) & (z < 4.0), f2, 0.0))
                else:
                    conv = f1
                terms.append(jnp.where(sels[c], wvs[c] * conv, 0.0))
            while len(terms) > 1:
                terms = [terms[i] + terms[i + 1] if i + 1 < len(terms)
                         else terms[i] for i in range(0, len(terms), 2)]
            return terms[0]

        acc = lax.cond(fastp, lambda: _accum(False), lambda: _accum(True))
        out_v[qs] = acc

    plsc.parallel_loop(0, NG, 1)(group_body)
    pltpu.sync_copy(out_v, out_hbm.at[pl.ds(base, QPW)])


@jax.jit
def kernel(x, weights, control_points):
    x0 = x[:, 0]
    x1 = x[:, 1]
    wf = weights[:, 0]
    lin = control_points[:N, 1]
    mesh = plsc.VectorSubcoreMesh(core_axis_name="c", subcore_axis_name="s")
    fn = pl.kernel(
        _body,
        out_type=jax.ShapeDtypeStruct((B,), jnp.float32),
        mesh=mesh,
        compiler_params=pltpu.CompilerParams(needs_layout_passes=False),
        scratch_types=[
            pltpu.VMEM((QPW,), jnp.float32),
            pltpu.VMEM((QPW,), jnp.float32),
            pltpu.VMEM((N * N,), jnp.float32),
            pltpu.VMEM((N,), jnp.float32),
            pltpu.VMEM((N,), jnp.float32),
            pltpu.VMEM((N,), jnp.float32),
            pltpu.VMEM((QPW,), jnp.float32),
            pltpu.SemaphoreType.DMA,
        ],
    )
    out = fn(x0, x1, wf, lin)
    return (out[:, None], x)
